# baseline (speedup 1.0000x reference)
import functools

import jax
import jax.numpy as jnp
from jax import lax
from jax.experimental import pallas as pl
from jax.experimental.pallas import tpu as pltpu
from jax.experimental.pallas import tpu_sc as plsc

_NC = 2
_NS = 16
_CH = 256
_SUB = 128


def _sc_mesh():
    return plsc.VectorSubcoreMesh(core_axis_name="c", subcore_axis_name="s",
                                  num_cores=_NC, num_subcores=_NS)


def _gather_rows(table, idx, rows_per_w):
    np_, d = idx.shape[0], table.shape[1]

    @functools.partial(
        pl.kernel,
        out_type=jax.ShapeDtypeStruct((np_, d), jnp.float32),
        mesh=_sc_mesh(),
        scratch_types=[
            pltpu.VMEM((rows_per_w,), jnp.int32),
            pltpu.VMEM((rows_per_w, d), jnp.float32),
            pltpu.SemaphoreType.DMA,
        ],
        compiler_params=pltpu.CompilerParams(use_tc_tiling_on_sc=False),
    )
    def k(table_hbm, idx_hbm, out_hbm, idx_v, rows_v, sem):
        wid = lax.axis_index("s") * _NC + lax.axis_index("c")
        base = wid * rows_per_w
        pltpu.sync_copy(idx_hbm.at[pl.ds(base, rows_per_w)], idx_v)
        pltpu.async_copy(table_hbm.at[idx_v], rows_v, sem).wait()
        pltpu.sync_copy(rows_v, out_hbm.at[pl.ds(base, rows_per_w)])

    return k(table, idx)


def _acc_rows(n_nodes):
    rn = 2 * n_nodes
    zr = -(-(rn + 1) // (8 * _NS)) * 8
    return rn, zr, zr * _NS


_GRP = 2048


def _seg_sums(table2, gidx, sidx2, z64, n_nodes):
    ep = gidx.shape[0] // _NC
    et = ep // _NS
    ngrp = et // _GRP
    rn, zr, srows = _acc_rows(n_nodes)

    def body(tab, gx, sx, z64r, s_out, shd, gidx_v, sidx_v, rows_b, rows_f,
             gsem, ssem):
        cid = lax.axis_index("c")
        sid = lax.axis_index("s")
        pltpu.sync_copy(z64r, shd.at[pl.ds(sid * zr, zr)])
        plsc.subcore_barrier()

        base = sid * et

        nck = _GRP // _CH

        def gather(k, buf):
            return pltpu.async_copy(
                tab.at[gidx_v.at[pl.ds(k * _CH, _CH)]],
                rows_b.at[buf], gsem)

        def unpack_chunk(buf):
            def rows8(r8, carry):
                r0 = r8 * 8
                for dr in range(8):
                    r = r0 + dr
                    for c in range(2):
                        ab = rows_b[buf, r, pl.ds(32 * c, 32)]
                        lo, hi = plsc.unpack(
                            ab, format=plsc.PackFormat.INTERLEAVED)
                        rows_f[r, pl.ds(32 * c, 16)] = lo
                        rows_f[r, pl.ds(32 * c + 16, 16)] = hi
                return carry
            lax.fori_loop(0, _CH // 8, rows8, 0)

        def group(i, carry):
            off = base + i * _GRP
            srow = pl.multiple_of(sid * (et // _CH) + i * (_GRP // _CH), 8)
            pltpu.sync_copy(gx.at[pl.ds(cid * ep + off, _GRP)], gidx_v)
            pltpu.sync_copy(sx.at[pl.ds(srow, _GRP // _CH)], sidx_v)
            gds = [gather(0, 0)] + [None] * (nck - 1)
            sds = [None] * nck
            for k in range(nck):
                gds[k].wait()
                if k + 1 < nck:
                    gds[k + 1] = gather(k + 1, (k + 1) % 2)
                if k >= 1:
                    sds[k - 1].wait()
                unpack_chunk(k % 2)
                sds[k] = pltpu.async_copy(
                    rows_f, shd.at[sidx_v.at[k]], ssem, add=True)
            sds[nck - 1].wait()
            return carry

        lax.fori_loop(0, ngrp, group, 0)
        plsc.subcore_barrier()
        pltpu.sync_copy(shd.at[pl.ds(sid * zr, zr)],
                        s_out.at[cid, pl.ds(sid * zr, zr)])

    k = pl.kernel(
        body,
        out_type=jax.ShapeDtypeStruct((_NC, srows, 64), jnp.float32),
        mesh=_sc_mesh(),
        scratch_types=[
            pltpu.VMEM_SHARED((srows, 64), jnp.float32),
            pltpu.VMEM((_GRP,), jnp.int32),
            pltpu.VMEM((_GRP // _CH, _CH), jnp.int32),
            pltpu.VMEM((2, _CH, 64), jnp.bfloat16),
            pltpu.VMEM((_CH, 64), jnp.float32),
            pltpu.SemaphoreType.DMA,
            pltpu.SemaphoreType.DMA,
        ],
        compiler_params=pltpu.CompilerParams(use_tc_tiling_on_sc=False,
                                             needs_layout_passes=False))
    return k(table2, gidx, sidx2, z64)


def _deg_counts(sidx2, z16, e0, n_nodes):
    ep = sidx2.shape[0] * _SUB
    eh = ep // _NC
    et = eh // _NS
    ngrp = et // _GRP
    rn, zr, srows = _acc_rows(n_nodes)

    def body(sx, z16r, e0r, deg_out, degshd, sidx_v, e0_v):
        cid = lax.axis_index("c")
        sid = lax.axis_index("s")
        pltpu.sync_copy(z16r, degshd.at[pl.ds(sid * zr, zr)])
        pltpu.sync_copy(e0r, e0_v)
        plsc.subcore_barrier()

        rbase = cid * (eh // _SUB) + sid * (et // _SUB)

        def group(i, carry):
            srow = pl.multiple_of(rbase + i * (_GRP // _SUB), 8)
            pltpu.sync_copy(sx.at[pl.ds(srow, _GRP // _SUB)], sidx_v)
            for j in range(_GRP // _SUB):
                pltpu.sync_copy(e0_v, degshd.at[sidx_v.at[j]], add=True)
            return carry

        lax.fori_loop(0, ngrp, group, 0)
        plsc.subcore_barrier()
        pltpu.sync_copy(degshd.at[pl.ds(sid * zr, zr)],
                        deg_out.at[cid, pl.ds(sid * zr, zr)])

    k = pl.kernel(
        body,
        out_type=jax.ShapeDtypeStruct((_NC, srows, 16), jnp.float32),
        mesh=_sc_mesh(),
        scratch_types=[
            pltpu.VMEM_SHARED((srows, 16), jnp.float32),
            pltpu.VMEM((_GRP // _SUB, _SUB), jnp.int32),
            pltpu.VMEM((_SUB, 16), jnp.float32),
        ],
        compiler_params=pltpu.CompilerParams(use_tc_tiling_on_sc=False))
    return k(sidx2, z16, e0)


def _combine(x, s, w3, deg_or_inv, n, relu, compute_inv):
    br = 2000
    g = n // br

    def bodyA(x_ref, s_ref, w_ref, d_ref, h_ref, inv_ref):
        deg = (jnp.sum(d_ref[0, 0], axis=1) + jnp.sum(d_ref[0, 1], axis=1)
               + jnp.sum(d_ref[1, 0], axis=1) + jnp.sum(d_ref[1, 1], axis=1))
        inv = 1.0 / jnp.maximum(deg, 1.0)
        inv_ref[...] = inv[:, None]
        acc = (jnp.dot(s_ref[0], w_ref[0], preferred_element_type=jnp.float32)
               + jnp.dot(s_ref[1], w_ref[1], preferred_element_type=jnp.float32))
        h = acc * inv[:, None] + jnp.dot(x_ref[...], w_ref[2],
                                         preferred_element_type=jnp.float32)
        h_ref[...] = jnp.maximum(h, 0.0) if relu else h

    def bodyB(x_ref, s_ref, w_ref, inv_ref, h_ref):
        inv = inv_ref[...]
        acc = (jnp.dot(s_ref[0], w_ref[0], preferred_element_type=jnp.float32)
               + jnp.dot(s_ref[1], w_ref[1], preferred_element_type=jnp.float32))
        h = acc * inv + jnp.dot(x_ref[...], w_ref[2],
                                preferred_element_type=jnp.float32)
        h_ref[...] = jnp.maximum(h, 0.0) if relu else h

    x_spec = pl.BlockSpec((br, 128), lambda i: (i, 0))
    s_spec = pl.BlockSpec((2, br, 128), lambda i: (0, i, 0))
    w_spec = pl.BlockSpec((3, 128, 128), lambda i: (0, 0, 0))
    h_spec = pl.BlockSpec((br, 128), lambda i: (i, 0))
    inv_spec = pl.BlockSpec((br, 1), lambda i: (i, 0))
    if compute_inv:
        d_spec = pl.BlockSpec((2, 2, br, 16), lambda i: (0, 0, i, 0))
        return pl.pallas_call(
            bodyA,
            grid=(g,),
            in_specs=[x_spec, s_spec, w_spec, d_spec],
            out_specs=(h_spec, inv_spec),
            out_shape=(jax.ShapeDtypeStruct((n, 128), jnp.float32),
                       jax.ShapeDtypeStruct((n, 1), jnp.float32)),
        )(x, s, w3, deg_or_inv)
    return pl.pallas_call(
        bodyB,
        grid=(g,),
        in_specs=[x_spec, s_spec, w_spec, inv_spec],
        out_specs=h_spec,
        out_shape=jax.ShapeDtypeStruct((n, 128), jnp.float32),
    )(x, s, w3, deg_or_inv)


def _discr_vec(positive, w_dc_t, n):
    br = 2000
    g = n // br

    def body(p_ref, wt_ref, v_ref):
        @pl.when(pl.program_id(0) == 0)
        def _init():
            v_ref[...] = jnp.zeros_like(v_ref)

        v_ref[...] += jnp.sum(p_ref[...], axis=0, keepdims=True)

        @pl.when(pl.program_id(0) == pl.num_programs(0) - 1)
        def _fin():
            sm = v_ref[...] * (1.0 / n)
            summary = 1.0 / (1.0 + jnp.exp(-sm))
            v_ref[...] = jnp.dot(summary, wt_ref[...],
                                 preferred_element_type=jnp.float32)

    return pl.pallas_call(
        body,
        grid=(g,),
        in_specs=[pl.BlockSpec((br, 128), lambda i: (i, 0)),
                  pl.BlockSpec((128, 128), lambda i: (0, 0))],
        out_specs=pl.BlockSpec((1, 128), lambda i: (0, 0)),
        out_shape=jax.ShapeDtypeStruct((1, 128), jnp.float32),
    )(positive, w_dc_t)


def _softplus(x):
    return jnp.maximum(x, 0.0) + jnp.log1p(jnp.exp(-jnp.abs(x)))


def _loss_kernel(positive, negative, feats, v_row, w_dec, wfin, n):
    br = 2000
    g = n // br

    def body(p_ref, n_ref, f_ref, v_ref, wd_ref, wf_ref, o_ref):
        @pl.when(pl.program_id(0) == 0)
        def _init():
            o_ref[...] = jnp.zeros_like(o_ref)

        vcol = jnp.reshape(v_ref[...], (128, 1))
        pos_log = jnp.dot(p_ref[...], vcol, preferred_element_type=jnp.float32)
        neg_log = jnp.dot(n_ref[...], vcol, preferred_element_type=jnp.float32)
        a = jnp.sum(_softplus(-pos_log))
        b = jnp.sum(_softplus(neg_log))
        r = jnp.dot(p_ref[...], wd_ref[...],
                    preferred_element_type=jnp.float32) - f_ref[...]
        c = jnp.sum(r * r)
        lane = lax.broadcasted_iota(jnp.int32, (1, 128), 1)
        vec = (jnp.where(lane == 0, a, 0.0) + jnp.where(lane == 1, b, 0.0)
               + jnp.where(lane == 2, c, 0.0))
        o_ref[...] += vec

        @pl.when(pl.program_id(0) == pl.num_programs(0) - 1)
        def _fin():
            total = jnp.sum(o_ref[...] * wf_ref[...])
            o_ref[...] = jnp.zeros_like(o_ref) + total

    return pl.pallas_call(
        body,
        grid=(g,),
        in_specs=[pl.BlockSpec((br, 128), lambda i: (i, 0)),
                  pl.BlockSpec((br, 128), lambda i: (i, 0)),
                  pl.BlockSpec((br, 128), lambda i: (i, 0)),
                  pl.BlockSpec((1, 128), lambda i: (0, 0)),
                  pl.BlockSpec((128, 128), lambda i: (0, 0)),
                  pl.BlockSpec((1, 128), lambda i: (0, 0))],
        out_specs=pl.BlockSpec((1, 128), lambda i: (0, 0)),
        out_shape=jax.ShapeDtypeStruct((1, 128), jnp.float32),
    )(positive, negative, feats, v_row, w_dec, wfin)


def kernel(features, edge_index, etype, corrupt_idx, W_basis0, comb0, W_self0,
           W_basis1, comb1, W_self1, W_dc, W_dec):
    n, d = features.shape
    e = etype.shape[0]
    rn = 2 * n

    chunk_total = _NC * _NS * _GRP
    ep = ((e + chunk_total - 1) // chunk_total) * chunk_total
    pad = ep - e
    src = edge_index[0]
    dst = edge_index[1]
    gb = jnp.pad(src * 2, (0, pad))
    gidx = jnp.concatenate([gb, gb + 1])
    sidx = jnp.pad(etype * n + dst, (0, pad), constant_values=rn)
    sidx2 = sidx.reshape(ep // _CH, _CH)
    sidx2d = sidx.reshape(ep // _SUB, _SUB)

    zr = -(-(rn + 1) // (8 * _NS)) * 8
    z64 = jnp.zeros((zr, 64), jnp.float32)
    z16 = jnp.zeros((zr, 16), jnp.float32)
    e0 = jnp.zeros((_SUB, 16), jnp.float32).at[:, 0].set(1.0)

    wr0 = jnp.einsum('rb,bio->rio', comb0, W_basis0)
    w3_0 = jnp.concatenate([wr0, W_self0[None]], axis=0)
    wr1 = jnp.einsum('rb,bio->rio', comb1, W_basis1)
    w3_1 = jnp.concatenate([wr1, W_self1[None]], axis=0)

    rows_per_w = (n + 255) // 256 * 8
    npad = 32 * rows_per_w
    ci = jnp.pad(corrupt_idx, (0, npad - n))
    xneg = _gather_rows(features, ci, rows_per_w)[:n]

    idx16 = jnp.arange(16)
    inter = jnp.stack([idx16, idx16 + 16], axis=1).ravel()
    colperm = jnp.concatenate([32 * g + inter for g in range(4)])

    def bf16_table(x):
        return x[:, colperm].astype(jnp.bfloat16).reshape(rn, 64)

    tab_p = bf16_table(features)
    tab_n = bf16_table(xneg)

    def assemble(s_out):
        return jnp.concatenate([s_out[0, :rn], s_out[1, :rn]],
                               axis=-1).reshape(2, n, 128)

    deg16 = _deg_counts(sidx2d, z16, e0, n)
    deg16 = deg16[:, :rn].reshape(2, 2, n, 16)

    s_p0 = _seg_sums(tab_p, gidx, sidx2, z64, n)
    h_p, inv = _combine(features, assemble(s_p0), w3_0, deg16, n,
                        relu=True, compute_inv=True)

    s_n0 = _seg_sums(tab_n, gidx, sidx2, z64, n)
    h_n = _combine(xneg, assemble(s_n0), w3_0, inv, n,
                   relu=True, compute_inv=False)

    s_p1 = _seg_sums(bf16_table(h_p), gidx, sidx2, z64, n)
    positive = _combine(h_p, assemble(s_p1), w3_1, inv, n,
                        relu=False, compute_inv=False)
    s_n1 = _seg_sums(bf16_table(h_n), gidx, sidx2, z64, n)
    negative = _combine(h_n, assemble(s_n1), w3_1, inv, n,
                        relu=False, compute_inv=False)

    v_row = _discr_vec(positive, W_dc.T, n)
    wfin = jnp.array([[1.0 / n, 1.0 / n, 1.0 / (n * d)]], jnp.float32)
    wfin = jnp.pad(wfin, ((0, 0), (0, 125)))
    out = _loss_kernel(positive, negative, features, v_row, W_dec, wfin, n)
    loss = out[0, 0]
    return (loss, positive)


# speedup vs baseline: 1.0336x; 1.0336x over previous
import functools

import jax
import jax.numpy as jnp
from jax import lax
from jax.experimental import pallas as pl
from jax.experimental.pallas import tpu as pltpu
from jax.experimental.pallas import tpu_sc as plsc

_NC = 2
_NS = 16
_CH = 256
_SUB = 128


def _sc_mesh():
    return plsc.VectorSubcoreMesh(core_axis_name="c", subcore_axis_name="s",
                                  num_cores=_NC, num_subcores=_NS)


def _gather_rows(table, idx, rows_per_w):
    np_, d = idx.shape[0], table.shape[1]

    @functools.partial(
        pl.kernel,
        out_type=jax.ShapeDtypeStruct((np_, d), jnp.float32),
        mesh=_sc_mesh(),
        scratch_types=[
            pltpu.VMEM((rows_per_w,), jnp.int32),
            pltpu.VMEM((rows_per_w, d), jnp.float32),
            pltpu.SemaphoreType.DMA,
        ],
        compiler_params=pltpu.CompilerParams(use_tc_tiling_on_sc=False),
    )
    def k(table_hbm, idx_hbm, out_hbm, idx_v, rows_v, sem):
        wid = lax.axis_index("s") * _NC + lax.axis_index("c")
        base = wid * rows_per_w
        pltpu.sync_copy(idx_hbm.at[pl.ds(base, rows_per_w)], idx_v)
        pltpu.async_copy(table_hbm.at[idx_v], rows_v, sem).wait()
        pltpu.sync_copy(rows_v, out_hbm.at[pl.ds(base, rows_per_w)])

    return k(table, idx)


def _acc_rows(n_nodes):
    rn = 2 * n_nodes
    zr = -(-(rn + 1) // (8 * _NS)) * 8
    return rn, zr, zr * _NS


_GRP = 2048


def _seg_sums(table2, gidx, sidx2, z64, n_nodes):
    ep = gidx.shape[0] // _NC
    et = ep // _NS
    ngrp = et // _GRP
    rn, zr, srows = _acc_rows(n_nodes)

    nrw = _GRP // _CH

    def body(tab, gx, sx, z64r, s_out, shd, gidx_v, sidx_v, rows_b, rows_f,
             gsem, ssem, isem):
        cid = lax.axis_index("c")
        sid = lax.axis_index("s")

        base = sid * et

        def idx_fetch(i, slot):
            off = base + i * _GRP
            srow = pl.multiple_of(sid * (et // _CH) + i * nrw, 8)
            pltpu.async_copy(gx.at[pl.ds(cid * ep + off, _GRP)],
                             gidx_v.at[slot], isem)
            pltpu.async_copy(sx.at[pl.ds(srow, nrw)], sidx_v.at[slot], isem)

        def idx_drain(slot):
            pltpu.make_async_copy(gx.at[pl.ds(0, _GRP)],
                                  gidx_v.at[slot], isem).wait()
            pltpu.make_async_copy(sx.at[pl.ds(0, nrw)],
                                  sidx_v.at[slot], isem).wait()

        idx_fetch(0, 0)
        pltpu.sync_copy(z64r, shd.at[pl.ds(sid * zr, zr)])
        plsc.subcore_barrier()

        nck = _GRP // _CH

        def gather(slot, k, buf):
            return pltpu.async_copy(
                tab.at[gidx_v.at[slot, pl.ds(k * _CH, _CH)]],
                rows_b.at[buf], gsem)

        def unpack_chunk(buf):
            def rows8(r8, carry):
                r0 = r8 * 8
                for dr in range(8):
                    r = r0 + dr
                    for c in range(2):
                        ab = rows_b[buf, r, pl.ds(32 * c, 32)]
                        lo, hi = plsc.unpack(
                            ab, format=plsc.PackFormat.INTERLEAVED)
                        rows_f[r, pl.ds(32 * c, 16)] = lo
                        rows_f[r, pl.ds(32 * c + 16, 16)] = hi
                return carry
            lax.fori_loop(0, _CH // 8, rows8, 0)

        def group(i, carry):
            slot = lax.rem(i, 2)
            idx_drain(slot)

            @pl.when(i + 1 < ngrp)
            def _prefetch():
                idx_fetch(i + 1, lax.rem(i + 1, 2))

            gds = [gather(slot, 0, 0)] + [None] * (nck - 1)
            sds = [None] * nck
            for k in range(nck):
                gds[k].wait()
                if k + 1 < nck:
                    gds[k + 1] = gather(slot, k + 1, (k + 1) % 2)
                if k >= 1:
                    sds[k - 1].wait()
                unpack_chunk(k % 2)
                sds[k] = pltpu.async_copy(
                    rows_f, shd.at[sidx_v.at[slot, k]], ssem, add=True)
            sds[nck - 1].wait()
            return carry

        lax.fori_loop(0, ngrp, group, 0)
        plsc.subcore_barrier()
        pltpu.sync_copy(shd.at[pl.ds(sid * zr, zr)],
                        s_out.at[cid, pl.ds(sid * zr, zr)])

    k = pl.kernel(
        body,
        out_type=jax.ShapeDtypeStruct((_NC, srows, 64), jnp.float32),
        mesh=_sc_mesh(),
        scratch_types=[
            pltpu.VMEM_SHARED((srows, 64), jnp.float32),
            pltpu.VMEM((2, _GRP), jnp.int32),
            pltpu.VMEM((2, _GRP // _CH, _CH), jnp.int32),
            pltpu.VMEM((2, _CH, 64), jnp.bfloat16),
            pltpu.VMEM((_CH, 64), jnp.float32),
            pltpu.SemaphoreType.DMA,
            pltpu.SemaphoreType.DMA,
            pltpu.SemaphoreType.DMA,
        ],
        compiler_params=pltpu.CompilerParams(use_tc_tiling_on_sc=False,
                                             needs_layout_passes=False))
    return k(table2, gidx, sidx2, z64)


def _deg_counts(sidx2, z16, e0, n_nodes):
    ep = sidx2.shape[0] * _SUB
    eh = ep // _NC
    et = eh // _NS
    ngrp = et // _GRP
    rn, zr, srows = _acc_rows(n_nodes)

    def body(sx, z16r, e0r, deg_out, degshd, sidx_v, e0_v):
        cid = lax.axis_index("c")
        sid = lax.axis_index("s")
        pltpu.sync_copy(z16r, degshd.at[pl.ds(sid * zr, zr)])
        pltpu.sync_copy(e0r, e0_v)
        plsc.subcore_barrier()

        rbase = cid * (eh // _SUB) + sid * (et // _SUB)

        def group(i, carry):
            srow = pl.multiple_of(rbase + i * (_GRP // _SUB), 8)
            pltpu.sync_copy(sx.at[pl.ds(srow, _GRP // _SUB)], sidx_v)
            for j in range(_GRP // _SUB):
                pltpu.sync_copy(e0_v, degshd.at[sidx_v.at[j]], add=True)
            return carry

        lax.fori_loop(0, ngrp, group, 0)
        plsc.subcore_barrier()
        pltpu.sync_copy(degshd.at[pl.ds(sid * zr, zr)],
                        deg_out.at[cid, pl.ds(sid * zr, zr)])

    k = pl.kernel(
        body,
        out_type=jax.ShapeDtypeStruct((_NC, srows, 16), jnp.float32),
        mesh=_sc_mesh(),
        scratch_types=[
            pltpu.VMEM_SHARED((srows, 16), jnp.float32),
            pltpu.VMEM((_GRP // _SUB, _SUB), jnp.int32),
            pltpu.VMEM((_SUB, 16), jnp.float32),
        ],
        compiler_params=pltpu.CompilerParams(use_tc_tiling_on_sc=False))
    return k(sidx2, z16, e0)


def _combine(x, s, w3, deg_or_inv, n, relu, compute_inv):
    br = 2000
    g = n // br

    def bodyA(x_ref, s_ref, w_ref, d_ref, h_ref, inv_ref):
        deg = (jnp.sum(d_ref[0, 0], axis=1) + jnp.sum(d_ref[0, 1], axis=1)
               + jnp.sum(d_ref[1, 0], axis=1) + jnp.sum(d_ref[1, 1], axis=1))
        inv = 1.0 / jnp.maximum(deg, 1.0)
        inv_ref[...] = inv[:, None]
        acc = (jnp.dot(s_ref[0], w_ref[0], preferred_element_type=jnp.float32)
               + jnp.dot(s_ref[1], w_ref[1], preferred_element_type=jnp.float32))
        h = acc * inv[:, None] + jnp.dot(x_ref[...], w_ref[2],
                                         preferred_element_type=jnp.float32)
        h_ref[...] = jnp.maximum(h, 0.0) if relu else h

    def bodyB(x_ref, s_ref, w_ref, inv_ref, h_ref):
        inv = inv_ref[...]
        acc = (jnp.dot(s_ref[0], w_ref[0], preferred_element_type=jnp.float32)
               + jnp.dot(s_ref[1], w_ref[1], preferred_element_type=jnp.float32))
        h = acc * inv + jnp.dot(x_ref[...], w_ref[2],
                                preferred_element_type=jnp.float32)
        h_ref[...] = jnp.maximum(h, 0.0) if relu else h

    x_spec = pl.BlockSpec((br, 128), lambda i: (i, 0))
    s_spec = pl.BlockSpec((2, br, 128), lambda i: (0, i, 0))
    w_spec = pl.BlockSpec((3, 128, 128), lambda i: (0, 0, 0))
    h_spec = pl.BlockSpec((br, 128), lambda i: (i, 0))
    inv_spec = pl.BlockSpec((br, 1), lambda i: (i, 0))
    if compute_inv:
        d_spec = pl.BlockSpec((2, 2, br, 16), lambda i: (0, 0, i, 0))
        return pl.pallas_call(
            bodyA,
            grid=(g,),
            in_specs=[x_spec, s_spec, w_spec, d_spec],
            out_specs=(h_spec, inv_spec),
            out_shape=(jax.ShapeDtypeStruct((n, 128), jnp.float32),
                       jax.ShapeDtypeStruct((n, 1), jnp.float32)),
        )(x, s, w3, deg_or_inv)
    return pl.pallas_call(
        bodyB,
        grid=(g,),
        in_specs=[x_spec, s_spec, w_spec, inv_spec],
        out_specs=h_spec,
        out_shape=jax.ShapeDtypeStruct((n, 128), jnp.float32),
    )(x, s, w3, deg_or_inv)


def _discr_vec(positive, w_dc_t, n):
    br = 2000
    g = n // br

    def body(p_ref, wt_ref, v_ref):
        @pl.when(pl.program_id(0) == 0)
        def _init():
            v_ref[...] = jnp.zeros_like(v_ref)

        v_ref[...] += jnp.sum(p_ref[...], axis=0, keepdims=True)

        @pl.when(pl.program_id(0) == pl.num_programs(0) - 1)
        def _fin():
            sm = v_ref[...] * (1.0 / n)
            summary = 1.0 / (1.0 + jnp.exp(-sm))
            v_ref[...] = jnp.dot(summary, wt_ref[...],
                                 preferred_element_type=jnp.float32)

    return pl.pallas_call(
        body,
        grid=(g,),
        in_specs=[pl.BlockSpec((br, 128), lambda i: (i, 0)),
                  pl.BlockSpec((128, 128), lambda i: (0, 0))],
        out_specs=pl.BlockSpec((1, 128), lambda i: (0, 0)),
        out_shape=jax.ShapeDtypeStruct((1, 128), jnp.float32),
    )(positive, w_dc_t)


def _softplus(x):
    return jnp.maximum(x, 0.0) + jnp.log1p(jnp.exp(-jnp.abs(x)))


def _loss_kernel(positive, negative, feats, v_row, w_dec, wfin, n):
    br = 2000
    g = n // br

    def body(p_ref, n_ref, f_ref, v_ref, wd_ref, wf_ref, o_ref):
        @pl.when(pl.program_id(0) == 0)
        def _init():
            o_ref[...] = jnp.zeros_like(o_ref)

        vcol = jnp.reshape(v_ref[...], (128, 1))
        pos_log = jnp.dot(p_ref[...], vcol, preferred_element_type=jnp.float32)
        neg_log = jnp.dot(n_ref[...], vcol, preferred_element_type=jnp.float32)
        a = jnp.sum(_softplus(-pos_log))
        b = jnp.sum(_softplus(neg_log))
        r = jnp.dot(p_ref[...], wd_ref[...],
                    preferred_element_type=jnp.float32) - f_ref[...]
        c = jnp.sum(r * r)
        lane = lax.broadcasted_iota(jnp.int32, (1, 128), 1)
        vec = (jnp.where(lane == 0, a, 0.0) + jnp.where(lane == 1, b, 0.0)
               + jnp.where(lane == 2, c, 0.0))
        o_ref[...] += vec

        @pl.when(pl.program_id(0) == pl.num_programs(0) - 1)
        def _fin():
            total = jnp.sum(o_ref[...] * wf_ref[...])
            o_ref[...] = jnp.zeros_like(o_ref) + total

    return pl.pallas_call(
        body,
        grid=(g,),
        in_specs=[pl.BlockSpec((br, 128), lambda i: (i, 0)),
                  pl.BlockSpec((br, 128), lambda i: (i, 0)),
                  pl.BlockSpec((br, 128), lambda i: (i, 0)),
                  pl.BlockSpec((1, 128), lambda i: (0, 0)),
                  pl.BlockSpec((128, 128), lambda i: (0, 0)),
                  pl.BlockSpec((1, 128), lambda i: (0, 0))],
        out_specs=pl.BlockSpec((1, 128), lambda i: (0, 0)),
        out_shape=jax.ShapeDtypeStruct((1, 128), jnp.float32),
    )(positive, negative, feats, v_row, w_dec, wfin)


def kernel(features, edge_index, etype, corrupt_idx, W_basis0, comb0, W_self0,
           W_basis1, comb1, W_self1, W_dc, W_dec):
    n, d = features.shape
    e = etype.shape[0]
    rn = 2 * n

    chunk_total = _NC * _NS * _GRP
    ep = ((e + chunk_total - 1) // chunk_total) * chunk_total
    pad = ep - e
    src = edge_index[0]
    dst = edge_index[1]
    gb = jnp.pad(src * 2, (0, pad))
    gidx = jnp.concatenate([gb, gb + 1])
    sidx = jnp.pad(etype * n + dst, (0, pad), constant_values=rn)
    sidx2 = sidx.reshape(ep // _CH, _CH)
    sidx2d = sidx.reshape(ep // _SUB, _SUB)

    zr = -(-(rn + 1) // (8 * _NS)) * 8
    z64 = jnp.zeros((zr, 64), jnp.float32)
    z16 = jnp.zeros((zr, 16), jnp.float32)
    e0 = jnp.zeros((_SUB, 16), jnp.float32).at[:, 0].set(1.0)

    wr0 = jnp.einsum('rb,bio->rio', comb0, W_basis0)
    w3_0 = jnp.concatenate([wr0, W_self0[None]], axis=0)
    wr1 = jnp.einsum('rb,bio->rio', comb1, W_basis1)
    w3_1 = jnp.concatenate([wr1, W_self1[None]], axis=0)

    rows_per_w = (n + 255) // 256 * 8
    npad = 32 * rows_per_w
    ci = jnp.pad(corrupt_idx, (0, npad - n))
    xneg = _gather_rows(features, ci, rows_per_w)[:n]

    idx16 = jnp.arange(16)
    inter = jnp.stack([idx16, idx16 + 16], axis=1).ravel()
    colperm = jnp.concatenate([32 * g + inter for g in range(4)])

    def bf16_table(x):
        return x[:, colperm].astype(jnp.bfloat16).reshape(rn, 64)

    tab_p = bf16_table(features)
    tab_n = bf16_table(xneg)

    def assemble(s_out):
        return jnp.concatenate([s_out[0, :rn], s_out[1, :rn]],
                               axis=-1).reshape(2, n, 128)

    deg16 = _deg_counts(sidx2d, z16, e0, n)
    deg16 = deg16[:, :rn].reshape(2, 2, n, 16)

    s_p0 = _seg_sums(tab_p, gidx, sidx2, z64, n)
    h_p, inv = _combine(features, assemble(s_p0), w3_0, deg16, n,
                        relu=True, compute_inv=True)

    s_n0 = _seg_sums(tab_n, gidx, sidx2, z64, n)
    h_n = _combine(xneg, assemble(s_n0), w3_0, inv, n,
                   relu=True, compute_inv=False)

    s_p1 = _seg_sums(bf16_table(h_p), gidx, sidx2, z64, n)
    positive = _combine(h_p, assemble(s_p1), w3_1, inv, n,
                        relu=False, compute_inv=False)
    s_n1 = _seg_sums(bf16_table(h_n), gidx, sidx2, z64, n)
    negative = _combine(h_n, assemble(s_n1), w3_1, inv, n,
                        relu=False, compute_inv=False)

    v_row = _discr_vec(positive, W_dc.T, n)
    wfin = jnp.array([[1.0 / n, 1.0 / n, 1.0 / (n * d)]], jnp.float32)
    wfin = jnp.pad(wfin, ((0, 0), (0, 125)))
    out = _loss_kernel(positive, negative, features, v_row, W_dec, wfin, n)
    loss = out[0, 0]
    return (loss, positive)


# speedup vs baseline: 1.0980x; 1.0623x over previous
import functools

import jax
import jax.numpy as jnp
from jax import lax
from jax.experimental import pallas as pl
from jax.experimental.pallas import tpu as pltpu
from jax.experimental.pallas import tpu_sc as plsc

_NC = 2
_NS = 16
_CH = 128
_SUB = 128


def _sc_mesh():
    return plsc.VectorSubcoreMesh(core_axis_name="c", subcore_axis_name="s",
                                  num_cores=_NC, num_subcores=_NS)


def _gather_rows(table, idx, rows_per_w):
    np_, d = idx.shape[0], table.shape[1]

    @functools.partial(
        pl.kernel,
        out_type=jax.ShapeDtypeStruct((np_, d), jnp.float32),
        mesh=_sc_mesh(),
        scratch_types=[
            pltpu.VMEM((rows_per_w,), jnp.int32),
            pltpu.VMEM((rows_per_w, d), jnp.float32),
            pltpu.SemaphoreType.DMA,
        ],
        compiler_params=pltpu.CompilerParams(use_tc_tiling_on_sc=False),
    )
    def k(table_hbm, idx_hbm, out_hbm, idx_v, rows_v, sem):
        wid = lax.axis_index("s") * _NC + lax.axis_index("c")
        base = wid * rows_per_w
        pltpu.sync_copy(idx_hbm.at[pl.ds(base, rows_per_w)], idx_v)
        pltpu.async_copy(table_hbm.at[idx_v], rows_v, sem).wait()
        pltpu.sync_copy(rows_v, out_hbm.at[pl.ds(base, rows_per_w)])

    return k(table, idx)


def _acc_rows(n_nodes):
    rn = 2 * n_nodes
    zr = -(-(rn + 1) // (8 * _NS)) * 8
    return rn, zr, zr * _NS


_GRP = 2048


def _seg_sums(table2, gidx, sidx2, z64, n_nodes):
    ep = gidx.shape[0] // _NC
    et = ep // _NS
    ngrp = et // _GRP
    rn, zr, srows = _acc_rows(n_nodes)

    nrw = _GRP // _CH

    def body(tab, gx, sx, z64r, s_out, shd, gidx_v, sidx_v, rows_b, rows_f,
             gsem, ssem, isem):
        cid = lax.axis_index("c")
        sid = lax.axis_index("s")

        base = sid * et

        def idx_fetch(i, slot):
            off = base + i * _GRP
            srow = pl.multiple_of(sid * (et // _CH) + i * nrw, 8)
            pltpu.async_copy(gx.at[pl.ds(cid * ep + off, _GRP)],
                             gidx_v.at[slot], isem)
            pltpu.async_copy(sx.at[pl.ds(srow, nrw)], sidx_v.at[slot], isem)

        def idx_drain(slot):
            pltpu.make_async_copy(gx.at[pl.ds(0, _GRP)],
                                  gidx_v.at[slot], isem).wait()
            pltpu.make_async_copy(sx.at[pl.ds(0, nrw)],
                                  sidx_v.at[slot], isem).wait()

        idx_fetch(0, 0)
        pltpu.sync_copy(z64r, shd.at[pl.ds(sid * zr, zr)])
        plsc.subcore_barrier()

        nck = _GRP // _CH

        def gather(slot, k, buf):
            return pltpu.async_copy(
                tab.at[gidx_v.at[slot, pl.ds(k * _CH, _CH)]],
                rows_b.at[buf], gsem)

        def unpack_chunk(buf, fbuf):
            def rows8(r8, carry):
                r0 = r8 * 8
                for dr in range(8):
                    r = r0 + dr
                    for c in range(2):
                        ab = rows_b[buf, r, pl.ds(32 * c, 32)]
                        lo, hi = plsc.unpack(
                            ab, format=plsc.PackFormat.INTERLEAVED)
                        rows_f[fbuf, r, pl.ds(32 * c, 16)] = lo
                        rows_f[fbuf, r, pl.ds(32 * c + 16, 16)] = hi
                return carry
            lax.fori_loop(0, _CH // 8, rows8, 0)

        def group(i, carry):
            slot = lax.rem(i, 2)
            idx_drain(slot)

            @pl.when(i + 1 < ngrp)
            def _prefetch():
                idx_fetch(i + 1, lax.rem(i + 1, 2))

            gds = [gather(slot, 0, 0)] + [None] * (nck - 1)
            sds = [None] * nck
            for k in range(nck):
                gds[k].wait()
                if k + 1 < nck:
                    gds[k + 1] = gather(slot, k + 1, (k + 1) % 2)
                if k >= 2:
                    sds[k - 2].wait()
                unpack_chunk(k % 2, k % 2)
                sds[k] = pltpu.async_copy(
                    rows_f.at[k % 2], shd.at[sidx_v.at[slot, k]], ssem,
                    add=True)
            sds[nck - 2].wait()
            sds[nck - 1].wait()
            return carry

        lax.fori_loop(0, ngrp, group, 0)
        plsc.subcore_barrier()
        pltpu.sync_copy(shd.at[pl.ds(sid * zr, zr)],
                        s_out.at[cid, pl.ds(sid * zr, zr)])

    k = pl.kernel(
        body,
        out_type=jax.ShapeDtypeStruct((_NC, srows, 64), jnp.float32),
        mesh=_sc_mesh(),
        scratch_types=[
            pltpu.VMEM_SHARED((srows, 64), jnp.float32),
            pltpu.VMEM((2, _GRP), jnp.int32),
            pltpu.VMEM((2, _GRP // _CH, _CH), jnp.int32),
            pltpu.VMEM((2, _CH, 64), jnp.bfloat16),
            pltpu.VMEM((2, _CH, 64), jnp.float32),
            pltpu.SemaphoreType.DMA,
            pltpu.SemaphoreType.DMA,
            pltpu.SemaphoreType.DMA,
        ],
        compiler_params=pltpu.CompilerParams(use_tc_tiling_on_sc=False,
                                             needs_layout_passes=False))
    return k(table2, gidx, sidx2, z64)


def _deg_counts(sidx2, z16, e0, n_nodes):
    ep = sidx2.shape[0] * _SUB
    eh = ep // _NC
    et = eh // _NS
    ngrp = et // _GRP
    rn, zr, srows = _acc_rows(n_nodes)

    def body(sx, z16r, e0r, deg_out, degshd, sidx_v, e0_v):
        cid = lax.axis_index("c")
        sid = lax.axis_index("s")
        pltpu.sync_copy(z16r, degshd.at[pl.ds(sid * zr, zr)])
        pltpu.sync_copy(e0r, e0_v)
        plsc.subcore_barrier()

        rbase = cid * (eh // _SUB) + sid * (et // _SUB)

        def group(i, carry):
            srow = pl.multiple_of(rbase + i * (_GRP // _SUB), 8)
            pltpu.sync_copy(sx.at[pl.ds(srow, _GRP // _SUB)], sidx_v)
            for j in range(_GRP // _SUB):
                pltpu.sync_copy(e0_v, degshd.at[sidx_v.at[j]], add=True)
            return carry

        lax.fori_loop(0, ngrp, group, 0)
        plsc.subcore_barrier()
        pltpu.sync_copy(degshd.at[pl.ds(sid * zr, zr)],
                        deg_out.at[cid, pl.ds(sid * zr, zr)])

    k = pl.kernel(
        body,
        out_type=jax.ShapeDtypeStruct((_NC, srows, 16), jnp.float32),
        mesh=_sc_mesh(),
        scratch_types=[
            pltpu.VMEM_SHARED((srows, 16), jnp.float32),
            pltpu.VMEM((_GRP // _SUB, _SUB), jnp.int32),
            pltpu.VMEM((_SUB, 16), jnp.float32),
        ],
        compiler_params=pltpu.CompilerParams(use_tc_tiling_on_sc=False))
    return k(sidx2, z16, e0)


def _combine(x, s, w3, deg_or_inv, n, relu, compute_inv):
    br = 2000
    g = n // br

    def bodyA(x_ref, s_ref, w_ref, d_ref, h_ref, inv_ref):
        deg = (jnp.sum(d_ref[0, 0], axis=1) + jnp.sum(d_ref[0, 1], axis=1)
               + jnp.sum(d_ref[1, 0], axis=1) + jnp.sum(d_ref[1, 1], axis=1))
        inv = 1.0 / jnp.maximum(deg, 1.0)
        inv_ref[...] = inv[:, None]
        acc = (jnp.dot(s_ref[0], w_ref[0], preferred_element_type=jnp.float32)
               + jnp.dot(s_ref[1], w_ref[1], preferred_element_type=jnp.float32))
        h = acc * inv[:, None] + jnp.dot(x_ref[...], w_ref[2],
                                         preferred_element_type=jnp.float32)
        h_ref[...] = jnp.maximum(h, 0.0) if relu else h

    def bodyB(x_ref, s_ref, w_ref, inv_ref, h_ref):
        inv = inv_ref[...]
        acc = (jnp.dot(s_ref[0], w_ref[0], preferred_element_type=jnp.float32)
               + jnp.dot(s_ref[1], w_ref[1], preferred_element_type=jnp.float32))
        h = acc * inv + jnp.dot(x_ref[...], w_ref[2],
                                preferred_element_type=jnp.float32)
        h_ref[...] = jnp.maximum(h, 0.0) if relu else h

    x_spec = pl.BlockSpec((br, 128), lambda i: (i, 0))
    s_spec = pl.BlockSpec((2, br, 128), lambda i: (0, i, 0))
    w_spec = pl.BlockSpec((3, 128, 128), lambda i: (0, 0, 0))
    h_spec = pl.BlockSpec((br, 128), lambda i: (i, 0))
    inv_spec = pl.BlockSpec((br, 1), lambda i: (i, 0))
    if compute_inv:
        d_spec = pl.BlockSpec((2, 2, br, 16), lambda i: (0, 0, i, 0))
        return pl.pallas_call(
            bodyA,
            grid=(g,),
            in_specs=[x_spec, s_spec, w_spec, d_spec],
            out_specs=(h_spec, inv_spec),
            out_shape=(jax.ShapeDtypeStruct((n, 128), jnp.float32),
                       jax.ShapeDtypeStruct((n, 1), jnp.float32)),
        )(x, s, w3, deg_or_inv)
    return pl.pallas_call(
        bodyB,
        grid=(g,),
        in_specs=[x_spec, s_spec, w_spec, inv_spec],
        out_specs=h_spec,
        out_shape=jax.ShapeDtypeStruct((n, 128), jnp.float32),
    )(x, s, w3, deg_or_inv)


def _discr_vec(positive, w_dc_t, n):
    br = 2000
    g = n // br

    def body(p_ref, wt_ref, v_ref):
        @pl.when(pl.program_id(0) == 0)
        def _init():
            v_ref[...] = jnp.zeros_like(v_ref)

        v_ref[...] += jnp.sum(p_ref[...], axis=0, keepdims=True)

        @pl.when(pl.program_id(0) == pl.num_programs(0) - 1)
        def _fin():
            sm = v_ref[...] * (1.0 / n)
            summary = 1.0 / (1.0 + jnp.exp(-sm))
            v_ref[...] = jnp.dot(summary, wt_ref[...],
                                 preferred_element_type=jnp.float32)

    return pl.pallas_call(
        body,
        grid=(g,),
        in_specs=[pl.BlockSpec((br, 128), lambda i: (i, 0)),
                  pl.BlockSpec((128, 128), lambda i: (0, 0))],
        out_specs=pl.BlockSpec((1, 128), lambda i: (0, 0)),
        out_shape=jax.ShapeDtypeStruct((1, 128), jnp.float32),
    )(positive, w_dc_t)


def _softplus(x):
    return jnp.maximum(x, 0.0) + jnp.log1p(jnp.exp(-jnp.abs(x)))


def _loss_kernel(positive, negative, feats, v_row, w_dec, wfin, n):
    br = 2000
    g = n // br

    def body(p_ref, n_ref, f_ref, v_ref, wd_ref, wf_ref, o_ref):
        @pl.when(pl.program_id(0) == 0)
        def _init():
            o_ref[...] = jnp.zeros_like(o_ref)

        vcol = jnp.reshape(v_ref[...], (128, 1))
        pos_log = jnp.dot(p_ref[...], vcol, preferred_element_type=jnp.float32)
        neg_log = jnp.dot(n_ref[...], vcol, preferred_element_type=jnp.float32)
        a = jnp.sum(_softplus(-pos_log))
        b = jnp.sum(_softplus(neg_log))
        r = jnp.dot(p_ref[...], wd_ref[...],
                    preferred_element_type=jnp.float32) - f_ref[...]
        c = jnp.sum(r * r)
        lane = lax.broadcasted_iota(jnp.int32, (1, 128), 1)
        vec = (jnp.where(lane == 0, a, 0.0) + jnp.where(lane == 1, b, 0.0)
               + jnp.where(lane == 2, c, 0.0))
        o_ref[...] += vec

        @pl.when(pl.program_id(0) == pl.num_programs(0) - 1)
        def _fin():
            total = jnp.sum(o_ref[...] * wf_ref[...])
            o_ref[...] = jnp.zeros_like(o_ref) + total

    return pl.pallas_call(
        body,
        grid=(g,),
        in_specs=[pl.BlockSpec((br, 128), lambda i: (i, 0)),
                  pl.BlockSpec((br, 128), lambda i: (i, 0)),
                  pl.BlockSpec((br, 128), lambda i: (i, 0)),
                  pl.BlockSpec((1, 128), lambda i: (0, 0)),
                  pl.BlockSpec((128, 128), lambda i: (0, 0)),
                  pl.BlockSpec((1, 128), lambda i: (0, 0))],
        out_specs=pl.BlockSpec((1, 128), lambda i: (0, 0)),
        out_shape=jax.ShapeDtypeStruct((1, 128), jnp.float32),
    )(positive, negative, feats, v_row, w_dec, wfin)


def kernel(features, edge_index, etype, corrupt_idx, W_basis0, comb0, W_self0,
           W_basis1, comb1, W_self1, W_dc, W_dec):
    n, d = features.shape
    e = etype.shape[0]
    rn = 2 * n

    chunk_total = _NC * _NS * _GRP
    ep = ((e + chunk_total - 1) // chunk_total) * chunk_total
    pad = ep - e
    src = edge_index[0]
    dst = edge_index[1]
    gb = jnp.pad(src * 2, (0, pad))
    gidx = jnp.concatenate([gb, gb + 1])
    sidx = jnp.pad(etype * n + dst, (0, pad), constant_values=rn)
    sidx2 = sidx.reshape(ep // _CH, _CH)
    sidx2d = sidx.reshape(ep // _SUB, _SUB)

    zr = -(-(rn + 1) // (8 * _NS)) * 8
    z64 = jnp.zeros((zr, 64), jnp.float32)
    z16 = jnp.zeros((zr, 16), jnp.float32)
    e0 = jnp.zeros((_SUB, 16), jnp.float32).at[:, 0].set(1.0)

    wr0 = jnp.einsum('rb,bio->rio', comb0, W_basis0)
    w3_0 = jnp.concatenate([wr0, W_self0[None]], axis=0)
    wr1 = jnp.einsum('rb,bio->rio', comb1, W_basis1)
    w3_1 = jnp.concatenate([wr1, W_self1[None]], axis=0)

    rows_per_w = (n + 255) // 256 * 8
    npad = 32 * rows_per_w
    ci = jnp.pad(corrupt_idx, (0, npad - n))
    xneg = _gather_rows(features, ci, rows_per_w)[:n]

    idx16 = jnp.arange(16)
    inter = jnp.stack([idx16, idx16 + 16], axis=1).ravel()
    colperm = jnp.concatenate([32 * g + inter for g in range(4)])

    def bf16_table(x):
        return x[:, colperm].astype(jnp.bfloat16).reshape(rn, 64)

    tab_p = bf16_table(features)
    tab_n = bf16_table(xneg)

    def assemble(s_out):
        return jnp.concatenate([s_out[0, :rn], s_out[1, :rn]],
                               axis=-1).reshape(2, n, 128)

    deg16 = _deg_counts(sidx2d, z16, e0, n)
    deg16 = deg16[:, :rn].reshape(2, 2, n, 16)

    s_p0 = _seg_sums(tab_p, gidx, sidx2, z64, n)
    h_p, inv = _combine(features, assemble(s_p0), w3_0, deg16, n,
                        relu=True, compute_inv=True)

    s_n0 = _seg_sums(tab_n, gidx, sidx2, z64, n)
    h_n = _combine(xneg, assemble(s_n0), w3_0, inv, n,
                   relu=True, compute_inv=False)

    s_p1 = _seg_sums(bf16_table(h_p), gidx, sidx2, z64, n)
    positive = _combine(h_p, assemble(s_p1), w3_1, inv, n,
                        relu=False, compute_inv=False)
    s_n1 = _seg_sums(bf16_table(h_n), gidx, sidx2, z64, n)
    negative = _combine(h_n, assemble(s_n1), w3_1, inv, n,
                        relu=False, compute_inv=False)

    v_row = _discr_vec(positive, W_dc.T, n)
    wfin = jnp.array([[1.0 / n, 1.0 / n, 1.0 / (n * d)]], jnp.float32)
    wfin = jnp.pad(wfin, ((0, 0), (0, 125)))
    out = _loss_kernel(positive, negative, features, v_row, W_dec, wfin, n)
    loss = out[0, 0]
    return (loss, positive)
